# baseline (speedup 1.0000x reference)
import functools

import jax
import jax.numpy as jnp
from jax import lax
from jax.experimental import pallas as pl
from jax.experimental.pallas import tpu as pltpu
from jax.experimental.pallas import tpu_sc as plsc

N = 10000
E = 320000
D = 128
H = 128
G = 128
NCLS = 10

NPAD = 10240
NB = NPAD // 128
NCORE = 2
NSUB = 16
NWORK = NCORE * NSUB
CH = 64
NCHUNK = 160
EW = NCHUNK * CH
EPAD = NWORK * EW
NBUF = 4
KDEG = 8
RPT = NPAD // NSUB

PREC = lax.Precision.HIGHEST
F32 = jnp.float32


def _mesh():
    return plsc.VectorSubcoreMesh(
        core_axis_name="c", subcore_axis_name="s",
        num_cores=NCORE, num_subcores=NSUB)


def _sc_degree_body(dst_hbm, z_hbm, out_hbm, didx, acc):
    cid = lax.axis_index("c")
    sid = lax.axis_index("s")
    w = cid * NSUB + sid
    pltpu.sync_copy(dst_hbm.at[pl.ds(w * EW, EW)], didx)
    pltpu.sync_copy(z_hbm, acc)
    ones16 = jnp.ones((16,), F32)

    def step(g, carry):
        for j in range(CH // 16):
            idxs = didx[pl.ds(g * CH + j * 16, 16)]
            plsc.addupdate_scatter(acc, [idxs], ones16)
        return carry

    lax.fori_loop(0, NCHUNK, step, 0)
    pltpu.sync_copy(acc, out_hbm.at[w])


_sc_degree = pl.kernel(
    _sc_degree_body,
    out_type=jax.ShapeDtypeStruct((NWORK, NPAD), F32),
    mesh=_mesh(),
    scratch_types=[
        pltpu.VMEM((EW,), jnp.int32),
        pltpu.VMEM((NPAD,), F32),
    ],
    compiler_params=pltpu.CompilerParams(needs_layout_passes=False),
)


def _sc_scatter_body(y_hbm, src_hbm, dst_hbm, z_hbm, out_hbm,
                     sidx0, sidx1, sidx2, sidx3,
                     didx0, didx1, didx2, didx3,
                     rows0, rows1, rows2, rows3, acc,
                     sem0, sem1, sem2, sem3):
    cid = lax.axis_index("c")
    sid = lax.axis_index("s")
    w = cid * NSUB + sid
    r0 = sid * RPT
    sidx = (sidx0, sidx1, sidx2, sidx3)
    didx = (didx0, didx1, didx2, didx3)
    rows = (rows0, rows1, rows2, rows3)
    sems = (sem0, sem1, sem2, sem3)
    pltpu.sync_copy(z_hbm, acc.at[pl.ds(r0, RPT)])
    plsc.subcore_barrier()

    def load_idx(b, g):
        base = w * EW + g * CH
        pltpu.sync_copy(src_hbm.at[pl.ds(base, CH)], sidx[b])
        pltpu.sync_copy(dst_hbm.at[pl.ds(base, CH)], didx[b])

    for b in range(NBUF):
        load_idx(b, b)
        pltpu.async_copy(y_hbm.at[sidx[b]], rows[b], sems[b])

    def outer(j, carry):
        for b in range(NBUF):
            g = j * NBUF + b
            pltpu.make_async_copy(y_hbm.at[sidx[b]], rows[b],
                                  sems[b]).wait()
            pltpu.sync_copy(rows[b], acc.at[didx[b]], add=True)
            gn = g + NBUF

            @pl.when(gn < NCHUNK)
            def _():
                load_idx(b, gn)
                pltpu.async_copy(y_hbm.at[sidx[b]], rows[b], sems[b])

        return carry

    lax.fori_loop(0, NCHUNK // NBUF, outer, 0)
    plsc.subcore_barrier()
    pltpu.sync_copy(acc.at[pl.ds(r0, RPT)], out_hbm.at[cid, pl.ds(r0, RPT)])


_sc_scatter = pl.kernel(
    _sc_scatter_body,
    out_type=jax.ShapeDtypeStruct((NCORE, NPAD, H), F32),
    mesh=_mesh(),
    scratch_types=(
        [pltpu.VMEM((CH,), jnp.int32)] * 8
        + [pltpu.VMEM((CH, H), F32)] * 4
        + [pltpu.VMEM_SHARED((NPAD, H), F32)]
        + [pltpu.SemaphoreType.DMA] * 4
    ),
)


def _col(row):
    return row.reshape(128, 1)


def _dgp_spec():
    return pl.BlockSpec((NWORK, 1, 1, 128), lambda i: (0, i, 0, 0))


def _dot(a, b):
    return jnp.dot(a, b, preferred_element_type=F32, precision=PREC)


def _row_spec(k):
    return pl.BlockSpec((128, k), lambda i: (i, 0))


def _deg_spec():
    return pl.BlockSpec((1, 1, 128), lambda i: (i, 0, 0))


def _whole(shape):
    nd = len(shape)
    return pl.BlockSpec(shape, lambda i: (0,) * nd)


def _part_spec():
    return pl.BlockSpec((NCORE, 128, H), lambda i: (0, i, 0))


def _t1_body(dgp_r, x_r, d0_r, w0_r, l0w_r, l0b_r, y0_r, lin0_r, dinv_r):
    dsum = jnp.sum(dgp_r[...], axis=0)[0]
    drow = lax.rsqrt(dsum + 1.0)
    dinv_r[...] = drow.reshape(1, 1, 128)
    dcol = _col(drow)
    y0_r[...] = _dot(x_r[...], w0_r[...]) * dcol
    lin0_r[...] = jnp.maximum(_dot(d0_r[...], l0w_r[...]) + l0b_r[...], 0.0)


def _t1(dgp, xp, d0p, w0, l0w, l0b):
    return pl.pallas_call(
        _t1_body,
        grid=(NB,),
        in_specs=[_dgp_spec(), _row_spec(D), _row_spec(D),
                  _whole((D, H)), _whole((D, H)), _whole((1, H))],
        out_specs=[_row_spec(H), _row_spec(H), _deg_spec()],
        out_shape=[jax.ShapeDtypeStruct((NPAD, H), F32),
                   jax.ShapeDtypeStruct((NPAD, H), F32),
                   jax.ShapeDtypeStruct((NB, 1, 128), F32)],
    )(dgp, xp, d0p, w0, l0w, l0b)


def _t2_body(p_r, y0_r, lin0_r, dv_r, d1_r, d2_r, b0_r, w1_r,
             l1w_r, l1b_r, y1_r, lin1_r):
    dcol = _col(dv_r[0])
    p = p_r[...]
    conv0 = jnp.maximum(dcol * (p[0] + p[1] + y0_r[...]) + b0_r[...], 0.0)
    comb = jnp.concatenate([lin0_r[...], conv0], axis=1)
    y1_r[...] = dcol * _dot(comb, w1_r[...])
    lf = jnp.concatenate([d1_r[...], d2_r[...]], axis=1)
    lin1_r[...] = jnp.maximum(_dot(lf, l1w_r[...]) + l1b_r[...], 0.0)


def _t2(p0, y0, lin0, dinvr, d1p, d2p, b0, w1, l1ws, l1b):
    return pl.pallas_call(
        _t2_body,
        grid=(NB,),
        in_specs=[_part_spec(), _row_spec(H), _row_spec(H),
                  _deg_spec(), _row_spec(D), _row_spec(D),
                  _whole((1, H)), _whole((2 * H, H)),
                  _whole((2 * D, H)), _whole((1, H))],
        out_specs=[_row_spec(H), _row_spec(H)],
        out_shape=[jax.ShapeDtypeStruct((NPAD, H), F32),
                   jax.ShapeDtypeStruct((NPAD, H), F32)],
    )(p0, y0, lin0, dinvr, d1p, d2p, b0, w1, l1ws, l1b)


def _t3_body(p_r, y1_r, lin1_r, dv_r, b1_r, wc_r, y2_r):
    dcol = _col(dv_r[0])
    p = p_r[...]
    conv1 = jnp.maximum(dcol * (p[0] + p[1] + y1_r[...]) + b1_r[...], 0.0)
    comb = jnp.concatenate([lin1_r[...], conv1], axis=1)
    y2_r[...] = dcol * _dot(comb, wc_r[...])


def _t3(p1, y1, lin1, dinvr, b1, wc):
    return pl.pallas_call(
        _t3_body,
        grid=(NB,),
        in_specs=[_part_spec(), _row_spec(H), _row_spec(H),
                  _deg_spec(), _whole((1, H)),
                  _whole((2 * H, H))],
        out_specs=_row_spec(H),
        out_shape=jax.ShapeDtypeStruct((NPAD, H), F32),
    )(p1, y1, lin1, dinvr, b1, wc)


def _t4_body(p_r, y2_r, dv_r, bat_r, bc_r, fcw_r, fcb_r, out_r, acc):
    i = pl.program_id(0)
    dcol = _col(dv_r[0])
    p = p_r[...]
    convx = dcol * (p[0] + p[1] + y2_r[...]) + bc_r[...]
    gcol = lax.broadcasted_iota(jnp.int32, (G, 1), 0)
    oh = (gcol == bat_r[0]).astype(F32)
    contrib = _dot(oh, convx)

    @pl.when(i == 0)
    def _():
        acc[...] = jnp.zeros_like(acc)

    acc[...] += contrib

    @pl.when(i == NB - 1)
    def _():
        out_r[...] = _dot(acc[...], fcw_r[...]) + fcb_r[...]


def _t4(p2, y2, dinvr, batp, bc, fcwp, fcbp):
    return pl.pallas_call(
        _t4_body,
        grid=(NB,),
        in_specs=[_part_spec(), _row_spec(H), _deg_spec(),
                  pl.BlockSpec((1, 1, 128), lambda i: (i, 0, 0)),
                  _whole((1, H)), _whole((H, 128)),
                  _whole((1, 128))],
        out_specs=_whole((G, 128)),
        out_shape=jax.ShapeDtypeStruct((G, 128), F32),
        scratch_shapes=[pltpu.VMEM((G, H), F32)],
    )(p2, y2, dinvr, batp, bc, fcwp, fcbp)


def kernel(x, edge_index, diff_feat, batch, conv0_W, conv0_b, conv1_W,
           conv1_b, lin0_W, lin0_b, lin1_W, lin1_b, order_weights, cls_W,
           cls_b, fc_W, fc_b):
    i32 = jnp.int32
    src = edge_index[0].astype(i32)
    dst = edge_index[1].astype(i32)
    pad_e = EPAD - E
    srcp = jnp.concatenate([src, jnp.full((pad_e,), N, i32)])
    dstp = jnp.concatenate([dst, jnp.full((pad_e,), N, i32)])

    pad_n = NPAD - N
    xp = jnp.pad(x, ((0, pad_n), (0, 0)))
    d0p = jnp.pad(diff_feat[0], ((0, pad_n), (0, 0)))
    d1p = jnp.pad(diff_feat[1], ((0, pad_n), (0, 0)))
    d2p = jnp.pad(diff_feat[2], ((0, pad_n), (0, 0)))
    batp = jnp.concatenate([batch.astype(i32),
                            jnp.full((pad_n,), G, i32)]).reshape(NB, 1, 128)

    zdeg = jnp.zeros((NPAD,), F32)
    zrows = jnp.zeros((RPT, H), F32)

    l1ws = jnp.concatenate([lin1_W[:D] * order_weights[0],
                            lin1_W[D:] * order_weights[1]])

    b0 = conv0_b.reshape(1, H)
    b1 = conv1_b.reshape(1, H)
    bcls = cls_b.reshape(1, H)
    l0b = lin0_b.reshape(1, H)
    l1b = lin1_b.reshape(1, H)
    fcwp = jnp.pad(fc_W, ((0, 0), (0, 128 - NCLS)))
    fcbp = jnp.pad(fc_b, (0, 128 - NCLS)).reshape(1, 128)

    degp = _sc_degree(dstp, zdeg)
    dgp = degp.reshape(NWORK, NB, 1, 128)

    y0, lin0, dinvr = _t1(dgp, xp, d0p, conv0_W, lin0_W, l0b)
    p0 = _sc_scatter(y0, srcp, dstp, zrows)
    y1, lin1 = _t2(p0, y0, lin0, dinvr, d1p, d2p, b0, conv1_W, l1ws, l1b)
    p1 = _sc_scatter(y1, srcp, dstp, zrows)
    y2 = _t3(p1, y1, lin1, dinvr, b1, cls_W)
    p2 = _sc_scatter(y2, srcp, dstp, zrows)
    out = _t4(p2, y2, dinvr, batp, bcls, fcwp, fcbp)
    return out[:, :NCLS]


# speedup vs baseline: 1.1889x; 1.1889x over previous
import functools

import jax
import jax.numpy as jnp
from jax import lax
from jax.experimental import pallas as pl
from jax.experimental.pallas import tpu as pltpu
from jax.experimental.pallas import tpu_sc as plsc

N = 10000
E = 320000
D = 128
H = 128
G = 128
NCLS = 10

NPAD = 10240
NB = NPAD // 128
NCORE = 2
NSUB = 16
NWORK = NCORE * NSUB
CH = 64
NCHUNK = 160
EW = NCHUNK * CH
EPAD = NWORK * EW
NBUF = 2
RPT = NPAD // NSUB

PREC = lax.Precision.DEFAULT
F32 = jnp.float32


def _mesh():
    return plsc.VectorSubcoreMesh(
        core_axis_name="c", subcore_axis_name="s",
        num_cores=NCORE, num_subcores=NSUB)


def _sc_degree_body(dst_hbm, z_hbm, out_hbm, didx, acc):
    cid = lax.axis_index("c")
    sid = lax.axis_index("s")
    w = cid * NSUB + sid
    pltpu.sync_copy(dst_hbm.at[pl.ds(w * EW, EW)], didx)
    pltpu.sync_copy(z_hbm, acc)
    ones16 = jnp.ones((16,), F32)

    def step(g, carry):
        for j in range(CH // 16):
            idxs = didx[pl.ds(g * CH + j * 16, 16)]
            plsc.addupdate_scatter(acc, [idxs], ones16)
        return carry

    lax.fori_loop(0, NCHUNK, step, 0)
    pltpu.sync_copy(acc, out_hbm.at[w])


_sc_degree = pl.kernel(
    _sc_degree_body,
    out_type=jax.ShapeDtypeStruct((NWORK, NPAD), F32),
    mesh=_mesh(),
    scratch_types=[
        pltpu.VMEM((EW,), jnp.int32),
        pltpu.VMEM((NPAD,), F32),
    ],
    compiler_params=pltpu.CompilerParams(needs_layout_passes=False),
)


def _sc_scatter_body(y_hbm, ei_hbm, z_hbm, out_hbm,
                     pk, sidx0, sidx1, didx0, didx1,
                     rows0, rows1, acc, sem0, sem1):
    cid = lax.axis_index("c")
    sid = lax.axis_index("s")
    w = cid * NSUB + sid
    r0 = sid * RPT
    sidx = (sidx0, sidx1)
    didx = (didx0, didx1)
    rows = (rows0, rows1)
    sems = (sem0, sem1)
    pltpu.sync_copy(ei_hbm.at[w], pk)
    pltpu.sync_copy(z_hbm, acc.at[pl.ds(r0, RPT)])
    plsc.subcore_barrier()

    mask = jnp.full((16,), 0xFFFF, jnp.int32)

    def unpack(b, g):
        for j in range(CH // 16):
            v = pk[g, pl.ds(j * 16, 16)]
            sidx[b][pl.ds(j * 16, 16)] = v & mask
            didx[b][pl.ds(j * 16, 16)] = lax.shift_right_logical(v, 16)

    for b in range(NBUF):
        unpack(b, b)
        pltpu.async_copy(y_hbm.at[sidx[b]], rows[b], sems[b])

    def outer(j, carry):
        for b in range(NBUF):
            g = j * NBUF + b
            pltpu.make_async_copy(y_hbm.at[sidx[b]], rows[b],
                                  sems[b]).wait()
            pltpu.sync_copy(rows[b], acc.at[didx[b]], add=True)
            gn = g + NBUF

            @pl.when(gn < NCHUNK)
            def _():
                unpack(b, gn)
                pltpu.async_copy(y_hbm.at[sidx[b]], rows[b], sems[b])

        return carry

    lax.fori_loop(0, NCHUNK // NBUF, outer, 0)
    plsc.subcore_barrier()
    pltpu.sync_copy(acc.at[pl.ds(r0, RPT)], out_hbm.at[cid, pl.ds(r0, RPT)])


_sc_scatter = pl.kernel(
    _sc_scatter_body,
    out_type=jax.ShapeDtypeStruct((NCORE, NPAD, H), F32),
    mesh=_mesh(),
    scratch_types=(
        [pltpu.VMEM((NCHUNK, CH), jnp.int32)]
        + [pltpu.VMEM((CH,), jnp.int32)] * 4
        + [pltpu.VMEM((CH, H), F32)] * 2
        + [pltpu.VMEM_SHARED((NPAD, H), F32)]
        + [pltpu.SemaphoreType.DMA] * 2
    ),
)


def _col(row):
    return row.reshape(128, 1)


def _dgp_spec():
    return pl.BlockSpec((NWORK, 1, 1, 128), lambda i: (0, i, 0, 0))


def _dot(a, b):
    return jnp.dot(a, b, preferred_element_type=F32, precision=PREC)


def _row_spec(k):
    return pl.BlockSpec((128, k), lambda i: (i, 0))


def _deg_spec():
    return pl.BlockSpec((1, 1, 128), lambda i: (i, 0, 0))


def _whole(shape):
    nd = len(shape)
    return pl.BlockSpec(shape, lambda i: (0,) * nd)


def _part_spec():
    return pl.BlockSpec((NCORE, 128, H), lambda i: (0, i, 0))


def _t1_body(dgp_r, x_r, d0_r, w0_r, l0w_r, l0b_r, y0_r, lin0_r, dinv_r):
    dsum = jnp.sum(dgp_r[...], axis=0)[0]
    drow = lax.rsqrt(dsum + 1.0)
    dinv_r[...] = drow.reshape(1, 1, 128)
    dcol = _col(drow)
    y0_r[...] = _dot(x_r[...], w0_r[...]) * dcol
    lin0_r[...] = jnp.maximum(_dot(d0_r[...], l0w_r[...]) + l0b_r[...], 0.0)


def _t1(dgp, xp, d0p, w0, l0w, l0b):
    return pl.pallas_call(
        _t1_body,
        grid=(NB,),
        in_specs=[_dgp_spec(), _row_spec(D), _row_spec(D),
                  _whole((D, H)), _whole((D, H)), _whole((1, H))],
        out_specs=[_row_spec(H), _row_spec(H), _deg_spec()],
        out_shape=[jax.ShapeDtypeStruct((NPAD, H), F32),
                   jax.ShapeDtypeStruct((NPAD, H), F32),
                   jax.ShapeDtypeStruct((NB, 1, 128), F32)],
    )(dgp, xp, d0p, w0, l0w, l0b)


def _t2_body(p_r, y0_r, lin0_r, dv_r, d1_r, d2_r, b0_r, w1_r,
             l1w_r, l1b_r, y1_r, lin1_r):
    dcol = _col(dv_r[0])
    p = p_r[...]
    conv0 = jnp.maximum(dcol * (p[0] + p[1] + y0_r[...]) + b0_r[...], 0.0)
    comb = jnp.concatenate([lin0_r[...], conv0], axis=1)
    y1_r[...] = dcol * _dot(comb, w1_r[...])
    lf = jnp.concatenate([d1_r[...], d2_r[...]], axis=1)
    lin1_r[...] = jnp.maximum(_dot(lf, l1w_r[...]) + l1b_r[...], 0.0)


def _t2(p0, y0, lin0, dinvr, d1p, d2p, b0, w1, l1ws, l1b):
    return pl.pallas_call(
        _t2_body,
        grid=(NB,),
        in_specs=[_part_spec(), _row_spec(H), _row_spec(H),
                  _deg_spec(), _row_spec(D), _row_spec(D),
                  _whole((1, H)), _whole((2 * H, H)),
                  _whole((2 * D, H)), _whole((1, H))],
        out_specs=[_row_spec(H), _row_spec(H)],
        out_shape=[jax.ShapeDtypeStruct((NPAD, H), F32),
                   jax.ShapeDtypeStruct((NPAD, H), F32)],
    )(p0, y0, lin0, dinvr, d1p, d2p, b0, w1, l1ws, l1b)


def _t3_body(p_r, y1_r, lin1_r, dv_r, b1_r, wc_r, y2_r):
    dcol = _col(dv_r[0])
    p = p_r[...]
    conv1 = jnp.maximum(dcol * (p[0] + p[1] + y1_r[...]) + b1_r[...], 0.0)
    comb = jnp.concatenate([lin1_r[...], conv1], axis=1)
    y2_r[...] = dcol * _dot(comb, wc_r[...])


def _t3(p1, y1, lin1, dinvr, b1, wc):
    return pl.pallas_call(
        _t3_body,
        grid=(NB,),
        in_specs=[_part_spec(), _row_spec(H), _row_spec(H),
                  _deg_spec(), _whole((1, H)),
                  _whole((2 * H, H))],
        out_specs=_row_spec(H),
        out_shape=jax.ShapeDtypeStruct((NPAD, H), F32),
    )(p1, y1, lin1, dinvr, b1, wc)


def _t4_body(p_r, y2_r, dv_r, bat_r, bc_r, fcw_r, fcb_r, out_r, acc):
    i = pl.program_id(0)
    dcol = _col(dv_r[0])
    p = p_r[...]
    convx = dcol * (p[0] + p[1] + y2_r[...]) + bc_r[...]
    gcol = lax.broadcasted_iota(jnp.int32, (G, 1), 0)
    oh = (gcol == bat_r[0]).astype(F32)
    contrib = _dot(oh, convx)

    @pl.when(i == 0)
    def _():
        acc[...] = jnp.zeros_like(acc)

    acc[...] += contrib

    @pl.when(i == NB - 1)
    def _():
        out_r[...] = _dot(acc[...], fcw_r[...]) + fcb_r[...]


def _t4(p2, y2, dinvr, batp, bc, fcwp, fcbp):
    return pl.pallas_call(
        _t4_body,
        grid=(NB,),
        in_specs=[_part_spec(), _row_spec(H), _deg_spec(),
                  pl.BlockSpec((1, 1, 128), lambda i: (i, 0, 0)),
                  _whole((1, H)), _whole((H, 128)),
                  _whole((1, 128))],
        out_specs=_whole((G, 128)),
        out_shape=jax.ShapeDtypeStruct((G, 128), F32),
        scratch_shapes=[pltpu.VMEM((G, H), F32)],
    )(p2, y2, dinvr, batp, bc, fcwp, fcbp)


def kernel(x, edge_index, diff_feat, batch, conv0_W, conv0_b, conv1_W,
           conv1_b, lin0_W, lin0_b, lin1_W, lin1_b, order_weights, cls_W,
           cls_b, fc_W, fc_b):
    i32 = jnp.int32
    src = edge_index[0].astype(i32)
    dst = edge_index[1].astype(i32)
    pad_e = EPAD - E
    srcp = jnp.concatenate([src, jnp.full((pad_e,), N, i32)])
    dstp = jnp.concatenate([dst, jnp.full((pad_e,), N, i32)])
    epk = (srcp | (dstp << 16)).reshape(NWORK, NCHUNK, CH)

    pad_n = NPAD - N
    xp = jnp.pad(x, ((0, pad_n), (0, 0)))
    d0p = jnp.pad(diff_feat[0], ((0, pad_n), (0, 0)))
    d1p = jnp.pad(diff_feat[1], ((0, pad_n), (0, 0)))
    d2p = jnp.pad(diff_feat[2], ((0, pad_n), (0, 0)))
    batp = jnp.concatenate([batch.astype(i32),
                            jnp.full((pad_n,), G, i32)]).reshape(NB, 1, 128)

    zdeg = jnp.zeros((NPAD,), F32)
    zrows = jnp.zeros((RPT, H), F32)

    l1ws = jnp.concatenate([lin1_W[:D] * order_weights[0],
                            lin1_W[D:] * order_weights[1]])

    b0 = conv0_b.reshape(1, H)
    b1 = conv1_b.reshape(1, H)
    bcls = cls_b.reshape(1, H)
    l0b = lin0_b.reshape(1, H)
    l1b = lin1_b.reshape(1, H)
    fcwp = jnp.pad(fc_W, ((0, 0), (0, 128 - NCLS)))
    fcbp = jnp.pad(fc_b, (0, 128 - NCLS)).reshape(1, 128)

    degp = _sc_degree(dstp, zdeg)
    dgp = degp.reshape(NWORK, NB, 1, 128)

    y0, lin0, dinvr = _t1(dgp, xp, d0p, conv0_W, lin0_W, l0b)
    p0 = _sc_scatter(y0, epk, zrows)
    y1, lin1 = _t2(p0, y0, lin0, dinvr, d1p, d2p, b0, conv1_W, l1ws, l1b)
    p1 = _sc_scatter(y1, epk, zrows)
    y2 = _t3(p1, y1, lin1, dinvr, b1, cls_W)
    p2 = _sc_scatter(y2, epk, zrows)
    out = _t4(p2, y2, dinvr, batp, bcls, fcwp, fcbp)
    return out[:, :NCLS]
